# Initial kernel scaffold; baseline (speedup 1.0000x reference)
import jax
import jax.numpy as jnp
from jax.experimental import pallas as pl


def kernel(token_tensor, table):
    raise NotImplementedError("write your pallas kernel here")


# speedup vs baseline: 1.0456x; 1.0456x over previous
import math

import jax
import jax.numpy as jnp
from jax import lax
from jax.experimental import pallas as pl
from jax.experimental.pallas import tpu as pltpu
from jax.experimental.pallas import tpu_sc as plsc

_D = 32
_SCALE = math.sqrt(float(_D))
_NC, _NS = 2, 16
_NW = _NC * _NS
_G = 128
_K = 4
_C = _G * _K
_L = 16


def _worker(table_hbm, idx_hbm, out_hbm, idx_v, buf0, buf1, gs0, gs1, ss0, ss1):
  n_idx_rows = idx_v.shape[0]
  n_sup = n_idx_rows // _K
  wid = lax.axis_index("s") * _NC + lax.axis_index("c")
  base = wid * n_idx_rows * _G

  pltpu.sync_copy(idx_hbm.at[wid], idx_v)

  def issue_gathers(s, buf, sem):
    for b in range(_K):
      pltpu.async_copy(table_hbm.at[idx_v.at[s * _K + b]],
                       buf.at[pl.ds(b * _G, _G)], sem)

  def drain_gathers(buf, sem):
    for b in range(_K):
      pltpu.make_async_copy(table_hbm.at[idx_v.at[0]],
                            buf.at[pl.ds(b * _G, _G)], sem).wait()

  def scale(buf):
    @pl.loop(0, _C, unroll=8)
    def _(i):
      buf[i, pl.ds(0, _L)] = buf[i, pl.ds(0, _L)] * _SCALE
      buf[i, pl.ds(_L, _L)] = buf[i, pl.ds(_L, _L)] * _SCALE

  def issue_scatter(s, buf, sem):
    pltpu.async_copy(buf, out_hbm.at[pl.ds(base + s * _C, _C)], sem)

  def wait_scatter(buf, sem):
    pltpu.make_async_copy(buf, out_hbm.at[pl.ds(base, _C)], sem).wait()

  issue_gathers(0, buf0, gs0)
  issue_gathers(1, buf1, gs1)

  @pl.loop(0, n_sup, step=2)
  def _(s):
    drain_gathers(buf0, gs0)
    scale(buf0)
    issue_scatter(s, buf0, ss0)

    drain_gathers(buf1, gs1)
    wait_scatter(buf0, ss0)

    @pl.when(s + 2 < n_sup)
    def _():
      issue_gathers(s + 2, buf0, gs0)

    scale(buf1)
    issue_scatter(s + 1, buf1, ss1)
    wait_scatter(buf1, ss1)

    @pl.when(s + 3 < n_sup)
    def _():
      issue_gathers(s + 3, buf1, gs1)


@jax.jit
def kernel(token_tensor, table):
  batch, seq = token_tensor.shape
  n_tok = batch * seq
  b_per_w = n_tok // _NW
  n_idx_rows = b_per_w // _G
  idx3 = token_tensor.astype(jnp.int32).reshape(_NW, n_idx_rows, _G)

  mesh = plsc.VectorSubcoreMesh(core_axis_name="c", subcore_axis_name="s")
  run = pl.kernel(
      _worker,
      out_type=jax.ShapeDtypeStruct((n_tok, _D), jnp.float32),
      mesh=mesh,
      compiler_params=pltpu.CompilerParams(use_tc_tiling_on_sc=False),
      scratch_types=[
          pltpu.VMEM((n_idx_rows, _G), jnp.int32),
          pltpu.VMEM((_C, _D), jnp.float32),
          pltpu.VMEM((_C, _D), jnp.float32),
          pltpu.SemaphoreType.DMA,
          pltpu.SemaphoreType.DMA,
          pltpu.SemaphoreType.DMA,
          pltpu.SemaphoreType.DMA,
      ],
  )
  out = run(table, idx3)
  return out.reshape(batch, seq, _D)


# speedup vs baseline: 1.0500x; 1.0042x over previous
import math

import jax
import jax.numpy as jnp
from jax import lax
from jax.experimental import pallas as pl
from jax.experimental.pallas import tpu as pltpu
from jax.experimental.pallas import tpu_sc as plsc

_D = 32
_SCALE = math.sqrt(float(_D))
_NC, _NS = 2, 16
_NW = _NC * _NS
_G = 128
_K = 4
_C = _G * _K
_L = 16
_NBUF = 5


def _worker(table_hbm, idx_hbm, out_hbm, *scratch):
  idx_v = scratch[0]
  bufs = scratch[1:1 + _NBUF]
  gsems = scratch[1 + _NBUF:1 + 2 * _NBUF]
  ssems = scratch[1 + 2 * _NBUF:1 + 3 * _NBUF]

  n_idx_rows = idx_v.shape[0]
  n_sup = n_idx_rows // _K
  wid = lax.axis_index("s") * _NC + lax.axis_index("c")
  base = wid * n_idx_rows * _G

  pltpu.sync_copy(idx_hbm.at[wid], idx_v)

  def issue_gathers(t, j):
    for b in range(_K):
      pltpu.async_copy(table_hbm.at[idx_v.at[t * _K + b]],
                       bufs[j].at[pl.ds(b * _G, _G)], gsems[j])

  def drain_gathers(j):
    for b in range(_K):
      pltpu.make_async_copy(table_hbm.at[idx_v.at[0]],
                            bufs[j].at[pl.ds(b * _G, _G)], gsems[j]).wait()

  def scale(j):
    buf = bufs[j]

    @pl.loop(0, _C, unroll=8)
    def _(i):
      buf[i, pl.ds(0, _L)] = buf[i, pl.ds(0, _L)] * _SCALE
      buf[i, pl.ds(_L, _L)] = buf[i, pl.ds(_L, _L)] * _SCALE

  def issue_scatter(t, j):
    pltpu.async_copy(bufs[j], out_hbm.at[pl.ds(base + t * _C, _C)], ssems[j])

  def wait_scatter(j):
    pltpu.make_async_copy(bufs[j], out_hbm.at[pl.ds(base, _C)],
                          ssems[j]).wait()

  for j in range(_NBUF - 1):
    issue_gathers(j, j)

  @pl.loop(0, n_sup, step=_NBUF)
  def _(s):
    for j in range(_NBUF):
      t = s + j
      drain_gathers(j)
      scale(j)
      issue_scatter(t, j)
      jn = (j + _NBUF - 1) % _NBUF

      @pl.when(t + _NBUF - 1 < n_sup)
      def _():
        @pl.when(t > 0)
        def _():
          wait_scatter(jn)

        issue_gathers(t + _NBUF - 1, jn)

  for j in range(_NBUF):
    wait_scatter(j)


@jax.jit
def kernel(token_tensor, table):
  batch, seq = token_tensor.shape
  n_tok = batch * seq
  b_per_w = n_tok // _NW
  n_idx_rows = b_per_w // _G
  idx3 = token_tensor.astype(jnp.int32).reshape(_NW, n_idx_rows, _G)

  mesh = plsc.VectorSubcoreMesh(core_axis_name="c", subcore_axis_name="s")
  scratch = [pltpu.VMEM((n_idx_rows, _G), jnp.int32)]
  scratch += [pltpu.VMEM((_C, _D), jnp.float32) for _ in range(_NBUF)]
  scratch += [pltpu.SemaphoreType.DMA for _ in range(2 * _NBUF)]
  run = pl.kernel(
      _worker,
      out_type=jax.ShapeDtypeStruct((n_tok, _D), jnp.float32),
      mesh=mesh,
      compiler_params=pltpu.CompilerParams(use_tc_tiling_on_sc=False),
      scratch_types=scratch,
  )
  out = run(table, idx3)
  return out.reshape(batch, seq, _D)


# speedup vs baseline: 1.4787x; 1.4083x over previous
import math

import jax
import jax.numpy as jnp
from jax import lax
from jax.experimental import pallas as pl
from jax.experimental.pallas import tpu as pltpu
from jax.experimental.pallas import tpu_sc as plsc

_D = 32
_SCALE = math.sqrt(float(_D))
_NC, _NS = 2, 16
_NW = _NC * _NS
_G = 128
_CB = 256
_GU = _CB // _G
_L = 16
_NBUF = 5


def _worker(table_hbm, idx_hbm, out_hbm, *scratch):
  idx_v = scratch[0]
  bufs = scratch[1:1 + _NBUF]
  bufTs = scratch[1 + _NBUF:1 + 2 * _NBUF]
  gsems = scratch[1 + 2 * _NBUF:1 + 3 * _NBUF]
  ssems = scratch[1 + 3 * _NBUF:1 + 4 * _NBUF]

  batch = out_hbm.shape[2]
  jb_per_s = batch // _CB
  n_units = idx_v.shape[0] // _GU
  wid = lax.axis_index("s") * _NC + lax.axis_index("c")
  u0 = wid * n_units
  iota = lax.iota(jnp.int32, _L)

  pltpu.sync_copy(idx_hbm.at[wid], idx_v)

  def issue_gathers(t, j):
    for g in range(_GU):
      pltpu.async_copy(table_hbm.at[idx_v.at[t * _GU + g]],
                       bufs[j].at[pl.ds(g * _G, _G)], gsems[j])

  def drain_gathers(j):
    for g in range(_GU):
      pltpu.make_async_copy(table_hbm.at[idx_v.at[0]],
                            bufs[j].at[pl.ds(g * _G, _G)], gsems[j]).wait()

  def transpose_scale(j):
    buf, bufT = bufs[j], bufTs[j]

    @pl.loop(0, _CB, unroll=4)
    def _(b):
      cols = jnp.full((_L,), b, jnp.int32)
      plsc.store_scatter(bufT, [iota, cols],
                         buf[b, pl.ds(0, _L)] * _SCALE)
      plsc.store_scatter(bufT, [iota + _L, cols],
                         buf[b, pl.ds(_L, _L)] * _SCALE)

  def issue_write(t, j):
    u = u0 + t
    s = u // jb_per_s
    jb = u % jb_per_s
    pltpu.async_copy(bufTs[j], out_hbm.at[s, :, pl.ds(jb * _CB, _CB)],
                     ssems[j])

  def wait_write(j):
    pltpu.make_async_copy(bufTs[j], out_hbm.at[0, :, pl.ds(0, _CB)],
                          ssems[j]).wait()

  for j in range(_NBUF - 1):
    issue_gathers(j, j)

  @pl.loop(0, n_units, step=_NBUF)
  def _(q):
    for j in range(_NBUF):
      t = q + j
      drain_gathers(j)
      transpose_scale(j)
      issue_write(t, j)
      jn = (j + _NBUF - 1) % _NBUF

      @pl.when(t + _NBUF - 1 < n_units)
      def _():
        @pl.when(t > 0)
        def _():
          wait_write(jn)

        issue_gathers(t + _NBUF - 1, jn)

  for j in range(_NBUF):
    wait_write(j)


@jax.jit
def kernel(token_tensor, table):
  batch, seq = token_tensor.shape
  n_tok = batch * seq
  n_idx_rows = n_tok // (_NW * _G)
  idx3 = token_tensor.T.astype(jnp.int32).reshape(_NW, n_idx_rows, _G)

  mesh = plsc.VectorSubcoreMesh(core_axis_name="c", subcore_axis_name="s")
  scratch = [pltpu.VMEM((n_idx_rows, _G), jnp.int32)]
  scratch += [pltpu.VMEM((_CB, _D), jnp.float32) for _ in range(_NBUF)]
  scratch += [pltpu.VMEM((_D, _CB), jnp.float32) for _ in range(_NBUF)]
  scratch += [pltpu.SemaphoreType.DMA for _ in range(2 * _NBUF)]
  run = pl.kernel(
      _worker,
      out_type=jax.ShapeDtypeStruct((seq, _D, batch), jnp.float32),
      mesh=mesh,
      compiler_params=pltpu.CompilerParams(use_tc_tiling_on_sc=False,
                                           needs_layout_passes=False),
      scratch_types=scratch,
  )
  out = run(table, idx3)
  return out.transpose(2, 0, 1)
